# Initial kernel scaffold; baseline (speedup 1.0000x reference)
import jax
import jax.numpy as jnp
from jax.experimental import pallas as pl


def kernel(x, edge_index, W1, b1, W2, b2, Wout, bout):
    raise NotImplementedError("write your pallas kernel here")


# speedup vs baseline: 4.3944x; 4.3944x over previous
import functools

import jax
import jax.numpy as jnp
from jax import lax
from jax.experimental import pallas as pl
from jax.experimental.pallas import tpu as pltpu
from jax.experimental.pallas import tpu_sc as plsc

N = 10000
E = 320000
D = 128
PE = 16
NC = 2
NT = 16
K = 128
NCHUNK = 79
EPT = NCHUNK * K
EPAD = NC * NT * EPT
NROW = N + 8
RCH = 80
NRC = N // RCH

ROW_BLOCK = 2000
GRID = N // ROW_BLOCK


def _mm_body(x_ref, w_ref, t_ref):
    t_ref[...] = jnp.dot(x_ref[...], w_ref[...],
                         preferred_element_type=jnp.float32)


def _mm(x, w):
    return pl.pallas_call(
        _mm_body,
        grid=(GRID,),
        in_specs=[
            pl.BlockSpec((ROW_BLOCK, D), lambda i: (i, 0)),
            pl.BlockSpec((D, D), lambda i: (0, 0)),
        ],
        out_specs=pl.BlockSpec((ROW_BLOCK, D), lambda i: (i, 0)),
        out_shape=jax.ShapeDtypeStruct((N, D), jnp.float32),
    )(x, w)


def _epi_mm_body(a0_ref, a1_ref, t_ref, b_ref, w_ref, h_ref, u_ref):
    h = a0_ref[...] + a1_ref[...] - t_ref[...] + b_ref[...]
    h = jnp.maximum(h, 0.0)
    h_ref[...] = h
    u_ref[...] = jnp.dot(h, w_ref[...], preferred_element_type=jnp.float32)


def _epi_mm(accs, t, b, w):
    return pl.pallas_call(
        _epi_mm_body,
        grid=(GRID,),
        in_specs=[
            pl.BlockSpec((ROW_BLOCK, D), lambda i: (i, 0)),
            pl.BlockSpec((ROW_BLOCK, D), lambda i: (i + GRID, 0)),
            pl.BlockSpec((ROW_BLOCK, D), lambda i: (i, 0)),
            pl.BlockSpec((1, D), lambda i: (0, 0)),
            pl.BlockSpec((D, D), lambda i: (0, 0)),
        ],
        out_specs=[
            pl.BlockSpec((ROW_BLOCK, D), lambda i: (i, 0)),
            pl.BlockSpec((ROW_BLOCK, D), lambda i: (i, 0)),
        ],
        out_shape=[
            jax.ShapeDtypeStruct((N, D), jnp.float32),
            jax.ShapeDtypeStruct((N, D), jnp.float32),
        ],
    )(accs, accs, t, b, w)


def _final_body(a0_ref, a1_ref, t_ref, b_ref, h1_ref, wout_ref, bout_ref,
                out_ref):
    x2 = a0_ref[...] + a1_ref[...] - t_ref[...] + b_ref[...]
    x2 = jnp.maximum(x2, 0.0) + h1_ref[...]
    out_ref[...] = (
        jnp.dot(x2, wout_ref[...], preferred_element_type=jnp.float32)
        + bout_ref[...]
    )


def _final(accs, t, b, h1, wout, bout):
    return pl.pallas_call(
        _final_body,
        grid=(GRID,),
        in_specs=[
            pl.BlockSpec((ROW_BLOCK, D), lambda i: (i, 0)),
            pl.BlockSpec((ROW_BLOCK, D), lambda i: (i + GRID, 0)),
            pl.BlockSpec((ROW_BLOCK, D), lambda i: (i, 0)),
            pl.BlockSpec((1, D), lambda i: (0, 0)),
            pl.BlockSpec((ROW_BLOCK, D), lambda i: (i, 0)),
            pl.BlockSpec((D, PE), lambda i: (0, 0)),
            pl.BlockSpec((1, PE), lambda i: (0, 0)),
        ],
        out_specs=pl.BlockSpec((ROW_BLOCK, PE), lambda i: (i, 0)),
        out_shape=jax.ShapeDtypeStruct((N, PE), jnp.float32),
    )(accs, accs, t, b, h1, wout, bout)


def _agg_body(t_hbm, src_hbm, dst_hbm, accs_out,
              src_b, dst_b, rows_v, acc_sh, sem_g0, sem_g1, sem_i0, sem_i1):
    c = lax.axis_index("c")
    s = lax.axis_index("s")
    base = (c * NT + s) * NCHUNK
    sems_g = (sem_g0, sem_g1)
    sems_i = (sem_i0, sem_i1)
    nmine = (NRC - 1 - s) // NT + 1

    def init_body(q, carry):
        r = (s + q * NT) * RCH
        pltpu.sync_copy(t_hbm.at[pl.ds(r, RCH)], acc_sh.at[pl.ds(r, RCH)])
        return carry

    lax.fori_loop(0, nmine, init_body, 0)
    plsc.subcore_barrier()

    def icopies(j, b):
        return (pltpu.make_async_copy(src_hbm.at[base + j], src_b.at[b],
                                      sems_i[b]),
                pltpu.make_async_copy(dst_hbm.at[base + j], dst_b.at[b],
                                      sems_i[b]))

    def istart(j, b):
        ca, cb = icopies(j, b)
        ca.start()
        cb.start()

    def iwait(j, b):
        ca, cb = icopies(j, b)
        ca.wait()
        cb.wait()

    def gcopy(b):
        return pltpu.make_async_copy(
            t_hbm.at[src_b.at[b]], rows_v.at[b], sems_g[b])

    def step(j, b):
        gcopy(b).wait()
        pltpu.sync_copy(rows_v.at[b], acc_sh.at[dst_b.at[b]], add=True)

        @pl.when(j + 2 < NCHUNK)
        def _():
            istart(j + 2, b)

        @pl.when(j + 1 < NCHUNK)
        def _():
            iwait(j + 1, 1 - b)
            gcopy(1 - b).start()

    istart(0, 0)
    istart(1, 1)
    iwait(0, 0)
    gcopy(0).start()

    def loop_body(i, carry):
        step(i * 2, 0)
        step(i * 2 + 1, 1)
        return carry

    lax.fori_loop(0, NCHUNK // 2, loop_body, 0)
    if NCHUNK % 2:
        step(NCHUNK - 1, 0)
    plsc.subcore_barrier()

    def drain_body(q, carry):
        r = (s + q * NT) * RCH
        pltpu.sync_copy(acc_sh.at[pl.ds(r, RCH)],
                        accs_out.at[pl.ds(c * N + r, RCH)])
        return carry

    lax.fori_loop(0, nmine, drain_body, 0)


_agg = functools.partial(
    pl.kernel,
    mesh=plsc.VectorSubcoreMesh(core_axis_name="c", subcore_axis_name="s"),
    out_type=jax.ShapeDtypeStruct((NC * N, D), jnp.float32),
    scratch_types=[
        pltpu.VMEM((2, K), jnp.int32),
        pltpu.VMEM((2, K), jnp.int32),
        pltpu.VMEM((2, K, D), jnp.float32),
        pltpu.VMEM_SHARED((NROW, D), jnp.float32),
        pltpu.SemaphoreType.DMA,
        pltpu.SemaphoreType.DMA,
        pltpu.SemaphoreType.DMA,
        pltpu.SemaphoreType.DMA,
    ],
)(_agg_body)


def kernel(x, edge_index, W1, b1, W2, b2, Wout, bout):
    pad = EPAD - E
    src = jnp.concatenate(
        [edge_index[0], jnp.zeros((pad,), jnp.int32)]).reshape(-1, K)
    dst = jnp.concatenate(
        [edge_index[1], jnp.full((pad,), N, jnp.int32)]).reshape(-1, K)
    t1 = _mm(x, W1)
    accs1 = _agg(t1, src, dst)
    h1, t2 = _epi_mm(accs1, t1, b1.reshape(1, D), W2)
    accs2 = _agg(t2, src, dst)
    return _final(accs2, t2, b2.reshape(1, D), h1, Wout, bout.reshape(1, PE))
